# baseline (speedup 1.0000x reference)
import functools

import jax
import jax.numpy as jnp
from jax import lax
from jax.experimental import pallas as pl
from jax.experimental.pallas import tpu as pltpu
from jax.experimental.pallas import tpu_sc as plsc

D_MODEL = 16
_NUM_WORKERS = 32
_CHUNK = 1600


def _build_gather(n_rows: int):
    b_per_w = n_rows // _NUM_WORKERS
    n_chunks = b_per_w // _CHUNK
    mesh = plsc.VectorSubcoreMesh(core_axis_name="c", subcore_axis_name="s")

    @functools.partial(
        pl.kernel,
        mesh=mesh,
        out_type=jax.ShapeDtypeStruct((n_rows, D_MODEL), jnp.float32),
        compiler_params=pltpu.CompilerParams(use_tc_tiling_on_sc=False),
        scratch_types=[
            pltpu.VMEM((_CHUNK,), jnp.int32),
            pltpu.VMEM((_CHUNK, D_MODEL), jnp.float32),
            pltpu.SemaphoreType.DMA,
        ],
    )
    def gather(idx_hbm, table_hbm, out_hbm, idx_v, rows_v, sem):
        wid = lax.axis_index("s") * 2 + lax.axis_index("c")
        base = wid * b_per_w

        def body(i, carry):
            off = base + i * _CHUNK
            pltpu.sync_copy(idx_hbm.at[pl.ds(off, _CHUNK)], idx_v)
            pltpu.async_copy(table_hbm.at[idx_v], rows_v, sem).wait()
            pltpu.sync_copy(rows_v, out_hbm.at[pl.ds(off, _CHUNK)])
            return carry

        lax.fori_loop(0, n_chunks, body, 0)

    return gather


@jax.jit
def kernel(tokens, embedding_weight):
    n_rows = tokens.shape[0] * tokens.shape[1]
    idx = tokens.reshape(-1).astype(jnp.int32)
    out = _build_gather(n_rows)(idx, embedding_weight)
    return out.reshape(tokens.shape[0], tokens.shape[1], D_MODEL)


# speedup vs baseline: 1.0180x; 1.0180x over previous
import functools

import jax
import jax.numpy as jnp
from jax import lax
from jax.experimental import pallas as pl
from jax.experimental.pallas import tpu as pltpu
from jax.experimental.pallas import tpu_sc as plsc

D_MODEL = 16
_NUM_WORKERS = 32
_CHUNK = 3200


def _build_gather(n_rows: int):
    b_per_w = n_rows // _NUM_WORKERS
    n_chunks = b_per_w // _CHUNK
    mesh = plsc.VectorSubcoreMesh(core_axis_name="c", subcore_axis_name="s")

    @functools.partial(
        pl.kernel,
        mesh=mesh,
        out_type=jax.ShapeDtypeStruct((n_rows, D_MODEL), jnp.float32),
        compiler_params=pltpu.CompilerParams(use_tc_tiling_on_sc=False),
        scratch_types=[
            pltpu.VMEM((b_per_w,), jnp.int32),
            pltpu.VMEM((2, _CHUNK, D_MODEL), jnp.float32),
            pltpu.SemaphoreType.DMA,
            pltpu.SemaphoreType.DMA,
        ],
    )
    def gather(idx_hbm, table_hbm, out_hbm, idx_v, rows_v, gsem, osem):
        wid = lax.axis_index("s") * 2 + lax.axis_index("c")
        base = wid * b_per_w

        pltpu.sync_copy(idx_hbm.at[pl.ds(base, b_per_w)], idx_v)

        def start_gather(g):
            return pltpu.async_copy(
                table_hbm.at[idx_v.at[pl.ds(g * _CHUNK, _CHUNK)]],
                rows_v.at[g % 2],
                gsem,
            )

        gat = start_gather(0)
        out = None
        for g in range(n_chunks):
            gat.wait()
            if out is not None:
                out.wait()
            if g + 1 < n_chunks:
                gat = start_gather(g + 1)
            out = pltpu.async_copy(
                rows_v.at[g % 2],
                out_hbm.at[pl.ds(base + g * _CHUNK, _CHUNK)],
                osem,
            )
        out.wait()

    return gather


@jax.jit
def kernel(tokens, embedding_weight):
    n_rows = tokens.shape[0] * tokens.shape[1]
    idx = tokens.reshape(-1).astype(jnp.int32)
    out = _build_gather(n_rows)(idx, embedding_weight)
    return out.reshape(tokens.shape[0], tokens.shape[1], D_MODEL)


# speedup vs baseline: 1.4633x; 1.4374x over previous
import functools

import jax
import jax.numpy as jnp
from jax import lax
from jax.experimental import pallas as pl
from jax.experimental.pallas import tpu as pltpu
from jax.experimental.pallas import tpu_sc as plsc

D_MODEL = 16
_NW = 32
_S = 200
_B = 4096
_RHO_PER_W = (_S * _B // 128) // _NW
_BATCH = 8
_NBATCH = _RHO_PER_W // _BATCH
_ROWS = _BATCH * 128


def _build_gather():
    mesh = plsc.VectorSubcoreMesh(core_axis_name="c", subcore_axis_name="s")
    n_out = _S * 2 * 32 * 8 * 128

    @functools.partial(
        pl.kernel,
        mesh=mesh,
        out_type=jax.ShapeDtypeStruct((n_out,), jnp.float32),
        compiler_params=pltpu.CompilerParams(
            use_tc_tiling_on_sc=False, needs_layout_passes=False
        ),
        scratch_types=[
            pltpu.VMEM((_RHO_PER_W * 128,), jnp.int32),
            pltpu.VMEM((2, _ROWS, D_MODEL), jnp.float32),
            pltpu.VMEM((2, 16 * 1024), jnp.float32),
            pltpu.SemaphoreType.DMA,
            pltpu.SemaphoreType.DMA,
            pltpu.SemaphoreType.DMA,
            pltpu.SemaphoreType.DMA,
        ],
    )
    def gather(idx_hbm, table_hbm, out_hbm, idx_v, rows_v, tbuf,
               gsem0, gsem1, osem0, osem1):
        gsems = (gsem0, gsem1)
        osems = (osem0, osem1)
        wid = lax.axis_index("s") * 2 + lax.axis_index("c")
        rho0 = wid * _RHO_PER_W

        pltpu.sync_copy(idx_hbm.at[pl.ds(rho0 * 128, _RHO_PER_W * 128)], idx_v)

        def start_gather(b):
            return pltpu.async_copy(
                table_hbm.at[idx_v.at[pl.ds(b * _ROWS, _ROWS)]],
                rows_v.at[b % 2],
                gsems[b % 2],
            )

        iota = lax.iota(jnp.int32, 16)

        def transpose_batch(b):
            s = b % 2
            rows_ref = rows_v.at[s]

            def step(t, carry):
                f = t >> 3
                l0 = (t & 7) * 16
                cols = (iota * 0 + f, iota * 0 + (8 + f))
                for s2 in range(8):
                    idx_row = iota + (s2 * 128) + l0
                    for dt in range(2):
                        val = plsc.load_gather(rows_ref, [idx_row, cols[dt]])
                        j = s2 * 2 + dt
                        tbuf[s, pl.ds(j * 1024 + f * 128 + l0, 16)] = val
                return carry

            lax.fori_loop(0, 64, step, 0)

        def start_writes(b):
            s = b % 2
            rho_b = rho0 + b * _BATCH
            s1 = rho_b // 256
            bt = (rho_b % 256) // 8
            ds = []
            for j in range(16):
                s2, dt = j >> 1, j & 1
                off = (((s1 * 8 + s2) * 2 + dt) * 32 + bt) * 1024
                ds.append(pltpu.async_copy(
                    tbuf.at[s, pl.ds(j * 1024, 1024)],
                    out_hbm.at[pl.ds(off, 1024)],
                    osems[s],
                ))
            return ds

        gat = {0: start_gather(0)}
        wdesc = {}
        for b in range(_NBATCH):
            if b >= 2:
                for d in wdesc[b - 2]:
                    d.wait()
            if b + 1 < _NBATCH:
                gat[b + 1] = start_gather(b + 1)
            gat[b].wait()
            transpose_batch(b)
            wdesc[b] = start_writes(b)
        for b in (_NBATCH - 2, _NBATCH - 1):
            for d in wdesc[b]:
                d.wait()

    return gather


@jax.jit
def kernel(tokens, embedding_weight):
    idx_phys = (
        tokens.T.astype(jnp.int32)
        .reshape(_S // 8, 8, _B // 128, 128)
        .transpose(0, 2, 1, 3)
        .reshape(-1)
    )
    out_flat = _build_gather()(idx_phys, embedding_weight)
    return (
        out_flat.reshape(_S, 2, _B // 128, 8, 128)
        .transpose(2, 4, 0, 1, 3)
        .reshape(_B, _S, D_MODEL)
    )
